# baseline (speedup 1.0000x reference)
import functools

import jax
import jax.numpy as jnp
from jax import lax
from jax.experimental import pallas as pl
from jax.experimental.pallas import tpu as pltpu
from jax.experimental.pallas import tpu_sc as plsc

T = 4096
D = 1024
E = 8
K = 2
H = 4096
RB = 512
NBLK = T // RB
BT = 256
NB = (T * K) // BT + (E - 1)
NBP = 48
B = NB * BT
BH = 512
NH = H // BH
NEG = -1e30


def _router_body(x_ref, wg_ref, bg_ref, wn_ref, bn_ref, eps_ref,
                 i0_ref, i1_ref, r0_ref, r1_ref, g0_ref, g1_ref, cnt_ref):
    x = x_ref[...]
    logits = jnp.dot(x, wg_ref[...], preferred_element_type=jnp.float32)
    logits = logits + bg_ref[...][None, :]
    nlin = jnp.dot(x, wn_ref[...], preferred_element_type=jnp.float32)
    nlin = nlin + bn_ref[...][None, :]
    sp = jnp.logaddexp(nlin, 0.0)
    noisy = logits + eps_ref[...] * sp

    iota_e = lax.broadcasted_iota(jnp.int32, (RB, E), 1)
    v0 = jnp.max(noisy, axis=-1, keepdims=True)
    i0 = jnp.min(jnp.where(noisy == v0, iota_e, E), axis=-1)
    masked = jnp.where(iota_e == i0[:, None], NEG, noisy)
    v1 = jnp.max(masked, axis=-1, keepdims=True)
    i1 = jnp.min(jnp.where(masked == v1, iota_e, E), axis=-1)

    e2 = jnp.exp(v1[:, 0] - v0[:, 0])
    g0 = 1.0 / (1.0 + e2)
    g1 = e2 / (1.0 + e2)

    oh0 = (iota_e == i0[:, None]).astype(jnp.float32)
    oh1 = (iota_e == i1[:, None]).astype(jnp.float32)
    s = oh0 + oh1
    rt = lax.broadcasted_iota(jnp.int32, (RB, RB), 0)
    ct = lax.broadcasted_iota(jnp.int32, (RB, RB), 1)
    tri = (ct < rt).astype(jnp.float32)
    run = jnp.dot(tri, s, preferred_element_type=jnp.float32)
    r0 = jnp.sum(run * oh0, axis=-1)
    r1 = jnp.sum(run * oh1, axis=-1)

    i0_ref[...] = i0.astype(jnp.int32)
    i1_ref[...] = i1.astype(jnp.int32)
    r0_ref[...] = r0.astype(jnp.int32)
    r1_ref[...] = r1.astype(jnp.int32)
    g0_ref[...] = g0
    g1_ref[...] = g1
    cnt_ref[...] = jnp.sum(s, axis=0).reshape(1, 1, E)


def _router(x, Wg, bg, Wn, bn, eps):
    vec = lambda: pl.BlockSpec((RB,), lambda b: (b,))
    return pl.pallas_call(
        _router_body,
        grid=(NBLK,),
        in_specs=[
            pl.BlockSpec((RB, D), lambda b: (b, 0)),
            pl.BlockSpec((D, E), lambda b: (0, 0)),
            pl.BlockSpec((E,), lambda b: (0,)),
            pl.BlockSpec((D, E), lambda b: (0, 0)),
            pl.BlockSpec((E,), lambda b: (0,)),
            pl.BlockSpec((RB, E), lambda b: (b, 0)),
        ],
        out_specs=[
            vec(), vec(), vec(), vec(), vec(), vec(),
            pl.BlockSpec((1, 1, E), lambda b: (b, 0, 0)),
        ],
        out_shape=[
            jax.ShapeDtypeStruct((T,), jnp.int32),
            jax.ShapeDtypeStruct((T,), jnp.int32),
            jax.ShapeDtypeStruct((T,), jnp.int32),
            jax.ShapeDtypeStruct((T,), jnp.int32),
            jax.ShapeDtypeStruct((T,), jnp.float32),
            jax.ShapeDtypeStruct((T,), jnp.float32),
            jax.ShapeDtypeStruct((NBLK, 1, E), jnp.float32),
        ],
    )(x, Wg, bg, Wn, bn, eps)


def _meta_body(cnt_ref, prefix_ref, be_ref):
    cnt = cnt_ref[...].reshape(NBLK, E)
    ones_row = jnp.ones((1, NBLK), jnp.float32)
    totals = jnp.dot(ones_row, cnt, preferred_element_type=jnp.float32)
    nblocks = jnp.floor((totals + (BT - 1)) * (1.0 / BT))
    rt = lax.broadcasted_iota(jnp.int32, (E, E), 0)
    ct = lax.broadcasted_iota(jnp.int32, (E, E), 1)
    tri_inc = (rt <= ct).astype(jnp.float32)
    endblk = jnp.dot(nblocks, tri_inc, preferred_element_type=jnp.float32)
    startblk = endblk - nblocks
    start = startblk * float(BT)

    rtb = lax.broadcasted_iota(jnp.int32, (NBLK, NBLK), 0)
    ctb = lax.broadcasted_iota(jnp.int32, (NBLK, NBLK), 1)
    tri_s = (ctb < rtb).astype(jnp.float32)
    run = jnp.dot(tri_s, cnt, preferred_element_type=jnp.float32)
    prefix_ref[...] = (start + run).astype(jnp.int32)

    g = lax.broadcasted_iota(jnp.int32, (NBP, E), 0).astype(jnp.float32)
    be = jnp.sum((endblk <= g).astype(jnp.int32), axis=-1)
    be_ref[...] = be


def _dispatch_meta(cnt):
    return pl.pallas_call(
        _meta_body,
        grid=(1,),
        in_specs=[pl.BlockSpec((NBLK, 1, E), lambda i: (0, 0, 0))],
        out_specs=[
            pl.BlockSpec((NBLK, E), lambda i: (0, 0)),
            pl.BlockSpec((NBP,), lambda i: (0,)),
        ],
        out_shape=[
            jax.ShapeDtypeStruct((NBLK, E), jnp.int32),
            jax.ShapeDtypeStruct((NBP,), jnp.int32),
        ],
    )(cnt)


def _sc_scatter(i0, i1, r0, r1, g0, g1, prefix_flat):
    mesh = plsc.VectorSubcoreMesh(core_axis_name="c", subcore_axis_name="s")

    @functools.partial(
        pl.kernel, mesh=mesh,
        out_type=[
            jax.ShapeDtypeStruct((B,), jnp.int32),
            jax.ShapeDtypeStruct((B,), jnp.float32),
            jax.ShapeDtypeStruct((T,), jnp.int32),
            jax.ShapeDtypeStruct((T,), jnp.int32),
        ],
        scratch_types=[
            pltpu.VMEM((B,), jnp.int32),
            pltpu.VMEM((B,), jnp.float32),
            pltpu.VMEM((T,), jnp.int32),
            pltpu.VMEM((T,), jnp.int32),
            pltpu.VMEM((T,), jnp.int32),
            pltpu.VMEM((T,), jnp.int32),
            pltpu.VMEM((T,), jnp.int32),
            pltpu.VMEM((T,), jnp.int32),
            pltpu.VMEM((T,), jnp.float32),
            pltpu.VMEM((T,), jnp.float32),
            pltpu.VMEM((NBLK * E,), jnp.int32),
        ],
        compiler_params=pltpu.CompilerParams(needs_layout_passes=False),
    )
    def k(i0_h, i1_h, r0_h, r1_h, g0_h, g1_h, pf_h,
          st_h, sg_h, p0_h, p1_h,
          st_v, sg_v, p0_v, p1_v, i0_v, i1_v, r0_v, r1_v, g0_v, g1_v, pf_v):
        wid = lax.axis_index("s") * 2 + lax.axis_index("c")

        @pl.when(wid == 0)
        def _():
            pltpu.sync_copy(i0_h, i0_v)
            pltpu.sync_copy(i1_h, i1_v)
            pltpu.sync_copy(r0_h, r0_v)
            pltpu.sync_copy(r1_h, r1_v)
            pltpu.sync_copy(g0_h, g0_v)
            pltpu.sync_copy(g1_h, g1_v)
            pltpu.sync_copy(pf_h, pf_v)

            zi = jnp.zeros((16,), jnp.int32)
            zf = jnp.zeros((16,), jnp.float32)

            def zero_body(c, _):
                st_v[pl.ds(c * 16, 16)] = zi
                sg_v[pl.ds(c * 16, 16)] = zf
                return 0
            lax.fori_loop(0, B // 16, zero_body, 0)

            lanes = lax.iota(jnp.int32, 16)

            def body(c, _):
                base = c * 16
                blk = c // (RB // 16)
                tok = base + lanes
                e0 = i0_v[pl.ds(base, 16)]
                e1 = i1_v[pl.ds(base, 16)]
                p0 = plsc.load_gather(pf_v, [blk * E + e0]) + r0_v[pl.ds(base, 16)]
                p1 = plsc.load_gather(pf_v, [blk * E + e1]) + r1_v[pl.ds(base, 16)]
                p0_v[pl.ds(base, 16)] = p0
                p1_v[pl.ds(base, 16)] = p1
                plsc.store_scatter(st_v, [p0], tok)
                plsc.store_scatter(st_v, [p1], tok)
                plsc.store_scatter(sg_v, [p0], g0_v[pl.ds(base, 16)])
                plsc.store_scatter(sg_v, [p1], g1_v[pl.ds(base, 16)])
                return 0
            lax.fori_loop(0, T // 16, body, 0)

            pltpu.sync_copy(st_v, st_h)
            pltpu.sync_copy(sg_v, sg_h)
            pltpu.sync_copy(p0_v, p0_h)
            pltpu.sync_copy(p1_v, p1_h)

    return k(i0, i1, r0, r1, g0, g1, prefix_flat)


def _sc_gather_x(x, sorted_tok):
    mesh = plsc.VectorSubcoreMesh(core_axis_name="c", subcore_axis_name="s")
    b_per_w = B // 32
    chunk = 104

    @functools.partial(
        pl.kernel, mesh=mesh,
        out_type=jax.ShapeDtypeStruct((B, D), jnp.float32),
        scratch_types=[
            pltpu.VMEM((chunk,), jnp.int32),
            pltpu.VMEM((chunk, D), jnp.float32),
            pltpu.SemaphoreType.DMA,
        ],
        compiler_params=pltpu.CompilerParams(needs_layout_passes=False),
    )
    def k(x_h, tok_h, out_h, idx_v, rows_v, sem):
        wid = lax.axis_index("s") * 2 + lax.axis_index("c")
        base = wid * b_per_w
        for c in range(b_per_w // chunk):
            pltpu.sync_copy(tok_h.at[pl.ds(base + c * chunk, chunk)], idx_v)
            pltpu.async_copy(x_h.at[idx_v], rows_v, sem).wait()
            pltpu.sync_copy(rows_v, out_h.at[pl.ds(base + c * chunk, chunk)])

    return k(x, sorted_tok)


def _grouped_body(be_ref, xs_ref, w1_ref, b1_ref, w2_ref, b2_ref, gate_ref,
                  out_ref):
    b = pl.program_id(0)
    h = pl.program_id(1)

    @pl.when(h == 0)
    def _():
        out_ref[...] = jnp.broadcast_to(b2_ref[0, 0], (BT, D))

    @pl.when(be_ref[b] < E)
    def _():
        ht = jnp.dot(xs_ref[...], w1_ref[0], preferred_element_type=jnp.float32)
        ht = jnp.maximum(ht + b1_ref[0, 0][None, :], 0.0)
        out_ref[...] += jnp.dot(ht, w2_ref[0], preferred_element_type=jnp.float32)

    @pl.when(h == NH - 1)
    def _():
        out_ref[...] *= gate_ref[0, 0][:, None]


def _grouped_ffn(block_expert, x_sorted, Wr1, br1, Wr2, br2, gate3):
    def emap(fn):
        return fn
    grid_spec = pltpu.PrefetchScalarGridSpec(
        num_scalar_prefetch=1,
        grid=(NB, NH),
        in_specs=[
            pl.BlockSpec((BT, D), lambda b, h, be: (b, 0)),
            pl.BlockSpec((1, D, BH), lambda b, h, be: (jnp.minimum(be[b], E - 1), 0, h)),
            pl.BlockSpec((1, 1, BH), lambda b, h, be: (jnp.minimum(be[b], E - 1), 0, h)),
            pl.BlockSpec((1, BH, D), lambda b, h, be: (jnp.minimum(be[b], E - 1), h, 0)),
            pl.BlockSpec((1, 1, D), lambda b, h, be: (jnp.minimum(be[b], E - 1), 0, 0)),
            pl.BlockSpec((1, 1, BT), lambda b, h, be: (b, 0, 0)),
        ],
        out_specs=pl.BlockSpec((BT, D), lambda b, h, be: (b, 0)),
    )
    return pl.pallas_call(
        _grouped_body,
        grid_spec=grid_spec,
        out_shape=jax.ShapeDtypeStruct((B, D), jnp.float32),
        compiler_params=pltpu.CompilerParams(
            dimension_semantics=("arbitrary", "arbitrary")),
    )(block_expert, x_sorted, Wr1, br1.reshape(E, 1, H), Wr2,
      br2.reshape(E, 1, D), gate3)


BTS = 512


def _shared_body(x_ref, w1_ref, b1_ref, w2_ref, b2_ref, out_ref):
    h = pl.program_id(1)

    @pl.when(h == 0)
    def _():
        out_ref[...] = jnp.broadcast_to(b2_ref[0], (BTS, D))

    ht = jnp.dot(x_ref[...], w1_ref[0], preferred_element_type=jnp.float32)
    ht = jnp.maximum(ht + b1_ref[0][None, :], 0.0)
    out_ref[...] += jnp.dot(ht, w2_ref[0], preferred_element_type=jnp.float32)


def _shared_ffn(x, Ws1, bs1, Ws2, bs2):
    return pl.pallas_call(
        _shared_body,
        grid=(T // BTS, NH),
        in_specs=[
            pl.BlockSpec((BTS, D), lambda i, h: (i, 0)),
            pl.BlockSpec((1, D, BH), lambda i, h: (0, 0, h)),
            pl.BlockSpec((1, BH), lambda i, h: (0, h)),
            pl.BlockSpec((1, BH, D), lambda i, h: (0, h, 0)),
            pl.BlockSpec((1, D), lambda i, h: (0, 0)),
        ],
        out_specs=pl.BlockSpec((BTS, D), lambda i, h: (i, 0)),
        out_shape=jax.ShapeDtypeStruct((T, D), jnp.float32),
        compiler_params=pltpu.CompilerParams(
            dimension_semantics=("arbitrary", "arbitrary")),
    )(x, Ws1, bs1, Ws2, bs2)


def _sc_combine(x, shared, out_sorted, pos0, pos1):
    mesh = plsc.VectorSubcoreMesh(core_axis_name="c", subcore_axis_name="s")
    tpw = T // 32
    CH = 16

    @functools.partial(
        pl.kernel, mesh=mesh,
        out_type=jax.ShapeDtypeStruct((T, D), jnp.float32),
        scratch_types=[
            pltpu.VMEM((tpw,), jnp.int32),
            pltpu.VMEM((tpw,), jnp.int32),
            pltpu.VMEM((CH, D), jnp.float32),
            pltpu.VMEM((CH, D), jnp.float32),
            pltpu.VMEM((CH, D), jnp.float32),
            pltpu.VMEM((CH, D), jnp.float32),
            pltpu.SemaphoreType.DMA,
        ],
        compiler_params=pltpu.CompilerParams(needs_layout_passes=False),
    )
    def k(x_h, sh_h, os_h, p0_h, p1_h, out_h,
          p0_v, p1_v, a_v, b_v, xs_v, ss_v, sem):
        wid = lax.axis_index("s") * 2 + lax.axis_index("c")
        tokbase = wid * tpw
        pltpu.sync_copy(p0_h.at[pl.ds(tokbase, tpw)], p0_v)
        pltpu.sync_copy(p1_h.at[pl.ds(tokbase, tpw)], p1_v)
        for c in range(tpw // CH):
            pltpu.async_copy(os_h.at[p0_v.at[pl.ds(c * CH, CH)]], a_v, sem).wait()
            pltpu.async_copy(os_h.at[p1_v.at[pl.ds(c * CH, CH)]], b_v, sem).wait()
            pltpu.sync_copy(x_h.at[pl.ds(tokbase + c * CH, CH)], xs_v)
            pltpu.sync_copy(sh_h.at[pl.ds(tokbase + c * CH, CH)], ss_v)

            def row(i, _):
                def col(j, _):
                    sl = pl.ds(j * 16, 16)
                    a_v[i, sl] = (a_v[i, sl] + b_v[i, sl]
                                  + xs_v[i, sl] + ss_v[i, sl])
                    return 0
                lax.fori_loop(0, D // 16, col, 0)
                return 0
            lax.fori_loop(0, CH, row, 0)
            pltpu.sync_copy(a_v, out_h.at[pl.ds(tokbase + c * CH, CH)])

    return k(x, shared, out_sorted, pos0, pos1)


def kernel(x, Wg, bg, Wn, bn, Wr1, br1, Wr2, br2, Ws1, bs1, Ws2, bs2):
    eps = jax.random.normal(jax.random.key(42), (T, E), dtype=jnp.float32)
    i0, i1, r0, r1, g0, g1, cnt = _router(x, Wg, bg, Wn, bn, eps)
    prefix, block_expert = _dispatch_meta(cnt)
    sorted_tok, sorted_gate, pos0, pos1 = _sc_scatter(
        i0, i1, r0, r1, g0, g1, prefix.reshape(NBLK * E))
    x_sorted = _sc_gather_x(x, sorted_tok)
    out_sorted = _grouped_ffn(block_expert, x_sorted, Wr1, br1, Wr2, br2,
                              sorted_gate.reshape(NB, 1, BT))
    shared = _shared_ffn(x, Ws1, bs1, Ws2, bs2)
    return _sc_combine(x, shared, out_sorted, pos0, pos1)


# speedup vs baseline: 1.1327x; 1.1327x over previous
import functools

import jax
import jax.numpy as jnp
from jax import lax
from jax.experimental import pallas as pl
from jax.experimental.pallas import tpu as pltpu
from jax.experimental.pallas import tpu_sc as plsc

T = 4096
D = 1024
E = 8
K = 2
H = 4096
RB = 512
NBLK = T // RB
BT = 512
NB = 24
NBP = 24
B = NB * BT
BH = 512
NH = H // BH
NEG = -1e30


def _router_body(x_ref, wg_ref, bg_ref, wn_ref, bn_ref, eps_ref,
                 i0_ref, i1_ref, r0_ref, r1_ref, g0_ref, g1_ref, cnt_ref):
    x = x_ref[...]
    logits = jnp.dot(x, wg_ref[...], preferred_element_type=jnp.float32)
    logits = logits + bg_ref[...][None, :]
    nlin = jnp.dot(x, wn_ref[...], preferred_element_type=jnp.float32)
    nlin = nlin + bn_ref[...][None, :]
    sp = jnp.logaddexp(nlin, 0.0)
    noisy = logits + eps_ref[...] * sp

    iota_e = lax.broadcasted_iota(jnp.int32, (RB, E), 1)
    v0 = jnp.max(noisy, axis=-1, keepdims=True)
    i0 = jnp.min(jnp.where(noisy == v0, iota_e, E), axis=-1)
    masked = jnp.where(iota_e == i0[:, None], NEG, noisy)
    v1 = jnp.max(masked, axis=-1, keepdims=True)
    i1 = jnp.min(jnp.where(masked == v1, iota_e, E), axis=-1)

    e2 = jnp.exp(v1[:, 0] - v0[:, 0])
    g0 = 1.0 / (1.0 + e2)
    g1 = e2 / (1.0 + e2)

    oh0 = (iota_e == i0[:, None]).astype(jnp.float32)
    oh1 = (iota_e == i1[:, None]).astype(jnp.float32)
    s = oh0 + oh1
    rt = lax.broadcasted_iota(jnp.int32, (RB, RB), 0)
    ct = lax.broadcasted_iota(jnp.int32, (RB, RB), 1)
    tri = (ct < rt).astype(jnp.float32)
    run = jnp.dot(tri, s, preferred_element_type=jnp.float32)
    r0 = jnp.sum(run * oh0, axis=-1)
    r1 = jnp.sum(run * oh1, axis=-1)

    i0_ref[...] = i0.astype(jnp.int32)
    i1_ref[...] = i1.astype(jnp.int32)
    r0_ref[...] = r0.astype(jnp.int32)
    r1_ref[...] = r1.astype(jnp.int32)
    g0_ref[...] = g0
    g1_ref[...] = g1
    cnt_ref[...] = jnp.sum(s, axis=0).reshape(1, 1, E)


def _router(x, Wg, bg, Wn, bn, eps):
    vec = lambda: pl.BlockSpec((RB,), lambda b: (b,))
    return pl.pallas_call(
        _router_body,
        grid=(NBLK,),
        in_specs=[
            pl.BlockSpec((RB, D), lambda b: (b, 0)),
            pl.BlockSpec((D, E), lambda b: (0, 0)),
            pl.BlockSpec((E,), lambda b: (0,)),
            pl.BlockSpec((D, E), lambda b: (0, 0)),
            pl.BlockSpec((E,), lambda b: (0,)),
            pl.BlockSpec((RB, E), lambda b: (b, 0)),
        ],
        out_specs=[
            vec(), vec(), vec(), vec(), vec(), vec(),
            pl.BlockSpec((1, 1, E), lambda b: (b, 0, 0)),
        ],
        out_shape=[
            jax.ShapeDtypeStruct((T,), jnp.int32),
            jax.ShapeDtypeStruct((T,), jnp.int32),
            jax.ShapeDtypeStruct((T,), jnp.int32),
            jax.ShapeDtypeStruct((T,), jnp.int32),
            jax.ShapeDtypeStruct((T,), jnp.float32),
            jax.ShapeDtypeStruct((T,), jnp.float32),
            jax.ShapeDtypeStruct((NBLK, 1, E), jnp.float32),
        ],
    )(x, Wg, bg, Wn, bn, eps)


def _meta_body(cnt_ref, prefix_ref, be_ref):
    cnt = cnt_ref[...].reshape(NBLK, E)
    ones_row = jnp.ones((1, NBLK), jnp.float32)
    totals = jnp.dot(ones_row, cnt, preferred_element_type=jnp.float32)
    nblocks = jnp.floor((totals + (BT - 1)) * (1.0 / BT))
    rt = lax.broadcasted_iota(jnp.int32, (E, E), 0)
    ct = lax.broadcasted_iota(jnp.int32, (E, E), 1)
    tri_inc = (rt <= ct).astype(jnp.float32)
    endblk = jnp.dot(nblocks, tri_inc, preferred_element_type=jnp.float32)
    startblk = endblk - nblocks
    start = startblk * float(BT)

    rtb = lax.broadcasted_iota(jnp.int32, (NBLK, NBLK), 0)
    ctb = lax.broadcasted_iota(jnp.int32, (NBLK, NBLK), 1)
    tri_s = (ctb < rtb).astype(jnp.float32)
    run = jnp.dot(tri_s, cnt, preferred_element_type=jnp.float32)
    prefix_ref[...] = (start + run).astype(jnp.int32)

    g = lax.broadcasted_iota(jnp.int32, (NBP, E), 0).astype(jnp.float32)
    be = jnp.sum((endblk <= g).astype(jnp.int32), axis=-1)
    be_ref[...] = be


def _dispatch_meta(cnt):
    return pl.pallas_call(
        _meta_body,
        grid=(1,),
        in_specs=[pl.BlockSpec((NBLK, 1, E), lambda i: (0, 0, 0))],
        out_specs=[
            pl.BlockSpec((NBLK, E), lambda i: (0, 0)),
            pl.BlockSpec((NBP,), lambda i: (0,)),
        ],
        out_shape=[
            jax.ShapeDtypeStruct((NBLK, E), jnp.int32),
            jax.ShapeDtypeStruct((NBP,), jnp.int32),
        ],
    )(cnt)


def _sc_scatter(i0, i1, r0, r1, g0, g1, prefix_flat):
    mesh = plsc.VectorSubcoreMesh(core_axis_name="c", subcore_axis_name="s")

    @functools.partial(
        pl.kernel, mesh=mesh,
        out_type=[
            jax.ShapeDtypeStruct((B,), jnp.int32),
            jax.ShapeDtypeStruct((B,), jnp.float32),
            jax.ShapeDtypeStruct((T,), jnp.int32),
            jax.ShapeDtypeStruct((T,), jnp.int32),
        ],
        scratch_types=[
            pltpu.VMEM((B,), jnp.int32),
            pltpu.VMEM((B,), jnp.float32),
            pltpu.VMEM((T,), jnp.int32),
            pltpu.VMEM((T,), jnp.int32),
            pltpu.VMEM((T,), jnp.int32),
            pltpu.VMEM((T,), jnp.int32),
            pltpu.VMEM((T,), jnp.int32),
            pltpu.VMEM((T,), jnp.int32),
            pltpu.VMEM((T,), jnp.float32),
            pltpu.VMEM((T,), jnp.float32),
            pltpu.VMEM((NBLK * E,), jnp.int32),
        ],
        compiler_params=pltpu.CompilerParams(needs_layout_passes=False),
    )
    def k(i0_h, i1_h, r0_h, r1_h, g0_h, g1_h, pf_h,
          st_h, sg_h, p0_h, p1_h,
          st_v, sg_v, p0_v, p1_v, i0_v, i1_v, r0_v, r1_v, g0_v, g1_v, pf_v):
        wid = lax.axis_index("s") * 2 + lax.axis_index("c")

        @pl.when(wid == 0)
        def _():
            pltpu.sync_copy(i0_h, i0_v)
            pltpu.sync_copy(i1_h, i1_v)
            pltpu.sync_copy(r0_h, r0_v)
            pltpu.sync_copy(r1_h, r1_v)
            pltpu.sync_copy(g0_h, g0_v)
            pltpu.sync_copy(g1_h, g1_v)
            pltpu.sync_copy(pf_h, pf_v)

            zi = jnp.zeros((16,), jnp.int32)
            zf = jnp.zeros((16,), jnp.float32)

            def zero_body(c, _):
                st_v[pl.ds(c * 16, 16)] = zi
                sg_v[pl.ds(c * 16, 16)] = zf
                return 0
            lax.fori_loop(0, B // 16, zero_body, 0)

            lanes = lax.iota(jnp.int32, 16)

            def body(c, _):
                base = c * 16
                blk = c // (RB // 16)
                tok = base + lanes
                e0 = i0_v[pl.ds(base, 16)]
                e1 = i1_v[pl.ds(base, 16)]
                p0 = plsc.load_gather(pf_v, [blk * E + e0]) + r0_v[pl.ds(base, 16)]
                p1 = plsc.load_gather(pf_v, [blk * E + e1]) + r1_v[pl.ds(base, 16)]
                p0_v[pl.ds(base, 16)] = p0
                p1_v[pl.ds(base, 16)] = p1
                plsc.store_scatter(st_v, [p0], tok)
                plsc.store_scatter(st_v, [p1], tok)
                plsc.store_scatter(sg_v, [p0], g0_v[pl.ds(base, 16)])
                plsc.store_scatter(sg_v, [p1], g1_v[pl.ds(base, 16)])
                return 0
            lax.fori_loop(0, T // 16, body, 0)

            pltpu.sync_copy(st_v, st_h)
            pltpu.sync_copy(sg_v, sg_h)
            pltpu.sync_copy(p0_v, p0_h)
            pltpu.sync_copy(p1_v, p1_h)

    return k(i0, i1, r0, r1, g0, g1, prefix_flat)


def _sc_gather_x(x, sorted_tok):
    mesh = plsc.VectorSubcoreMesh(core_axis_name="c", subcore_axis_name="s")
    b_per_w = B // 32
    chunk = 48
    nch = b_per_w // chunk

    @functools.partial(
        pl.kernel, mesh=mesh,
        out_type=jax.ShapeDtypeStruct((B, D), jnp.float32),
        scratch_types=[
            pltpu.VMEM((b_per_w,), jnp.int32),
            pltpu.VMEM((chunk, D), jnp.float32),
            pltpu.VMEM((chunk, D), jnp.float32),
            pltpu.SemaphoreType.DMA,
            pltpu.SemaphoreType.DMA,
        ],
        compiler_params=pltpu.CompilerParams(needs_layout_passes=False),
    )
    def k(x_h, tok_h, out_h, idx_v, r0_v, r1_v, sem0, sem1):
        wid = lax.axis_index("s") * 2 + lax.axis_index("c")
        base = wid * b_per_w
        pltpu.sync_copy(tok_h.at[pl.ds(base, b_per_w)], idx_v)
        bufs = (r0_v, r1_v)
        sems = (sem0, sem1)
        cps = [None, None]
        cps[0] = pltpu.async_copy(x_h.at[idx_v.at[pl.ds(0, chunk)]], r0_v, sem0)
        for c in range(nch):
            if c + 1 < nch:
                cps[(c + 1) % 2] = pltpu.async_copy(
                    x_h.at[idx_v.at[pl.ds((c + 1) * chunk, chunk)]],
                    bufs[(c + 1) % 2], sems[(c + 1) % 2])
            cps[c % 2].wait()
            pltpu.sync_copy(bufs[c % 2], out_h.at[pl.ds(base + c * chunk, chunk)])

    return k(x, sorted_tok)


def _grouped_body(be_ref, xs_ref, w1_ref, b1_ref, w2_ref, b2_ref, gate_ref,
                  out_ref):
    b = pl.program_id(0)
    h = pl.program_id(1)

    @pl.when(h == 0)
    def _():
        out_ref[...] = jnp.broadcast_to(b2_ref[0, 0], (BT, D))

    @pl.when(be_ref[b] < E)
    def _():
        ht = jnp.dot(xs_ref[...], w1_ref[0], preferred_element_type=jnp.float32)
        ht = jnp.maximum(ht + b1_ref[0, 0][None, :], 0.0)
        out_ref[...] += jnp.dot(ht, w2_ref[0], preferred_element_type=jnp.float32)

    @pl.when(h == NH - 1)
    def _():
        out_ref[...] *= gate_ref[0, 0][:, None]


def _grouped_ffn(block_expert, x_sorted, Wr1, br1, Wr2, br2, gate3):
    def emap(fn):
        return fn
    grid_spec = pltpu.PrefetchScalarGridSpec(
        num_scalar_prefetch=1,
        grid=(NB, NH),
        in_specs=[
            pl.BlockSpec((BT, D), lambda b, h, be: (b, 0)),
            pl.BlockSpec((1, D, BH), lambda b, h, be: (jnp.minimum(be[b], E - 1), 0, h)),
            pl.BlockSpec((1, 1, BH), lambda b, h, be: (jnp.minimum(be[b], E - 1), 0, h)),
            pl.BlockSpec((1, BH, D), lambda b, h, be: (jnp.minimum(be[b], E - 1), h, 0)),
            pl.BlockSpec((1, 1, D), lambda b, h, be: (jnp.minimum(be[b], E - 1), 0, 0)),
            pl.BlockSpec((1, 1, BT), lambda b, h, be: (b, 0, 0)),
        ],
        out_specs=pl.BlockSpec((BT, D), lambda b, h, be: (b, 0)),
    )
    return pl.pallas_call(
        _grouped_body,
        grid_spec=grid_spec,
        out_shape=jax.ShapeDtypeStruct((B, D), jnp.float32),
        compiler_params=pltpu.CompilerParams(
            dimension_semantics=("arbitrary", "arbitrary")),
    )(block_expert, x_sorted, Wr1, br1.reshape(E, 1, H), Wr2,
      br2.reshape(E, 1, D), gate3)


BTS = 512


def _shared_body(x_ref, w1_ref, b1_ref, w2_ref, b2_ref, out_ref):
    h = pl.program_id(1)

    @pl.when(h == 0)
    def _():
        out_ref[...] = jnp.broadcast_to(b2_ref[0], (BTS, D))

    ht = jnp.dot(x_ref[...], w1_ref[0], preferred_element_type=jnp.float32)
    ht = jnp.maximum(ht + b1_ref[0][None, :], 0.0)
    out_ref[...] += jnp.dot(ht, w2_ref[0], preferred_element_type=jnp.float32)


def _shared_ffn(x, Ws1, bs1, Ws2, bs2):
    return pl.pallas_call(
        _shared_body,
        grid=(T // BTS, NH),
        in_specs=[
            pl.BlockSpec((BTS, D), lambda i, h: (i, 0)),
            pl.BlockSpec((1, D, BH), lambda i, h: (0, 0, h)),
            pl.BlockSpec((1, BH), lambda i, h: (0, h)),
            pl.BlockSpec((1, BH, D), lambda i, h: (0, h, 0)),
            pl.BlockSpec((1, D), lambda i, h: (0, 0)),
        ],
        out_specs=pl.BlockSpec((BTS, D), lambda i, h: (i, 0)),
        out_shape=jax.ShapeDtypeStruct((T, D), jnp.float32),
        compiler_params=pltpu.CompilerParams(
            dimension_semantics=("arbitrary", "arbitrary")),
    )(x, Ws1, bs1, Ws2, bs2)


def _sc_combine(out_sorted, pos0, pos1):
    mesh = plsc.VectorSubcoreMesh(core_axis_name="c", subcore_axis_name="s")
    tpw = T // 32
    CH = 16
    nch = tpw // CH

    @functools.partial(
        pl.kernel, mesh=mesh,
        out_type=jax.ShapeDtypeStruct((T, D), jnp.float32),
        scratch_types=[
            pltpu.VMEM((tpw,), jnp.int32),
            pltpu.VMEM((tpw,), jnp.int32),
            pltpu.VMEM((CH, D), jnp.float32),
            pltpu.VMEM((CH, D), jnp.float32),
            pltpu.VMEM((CH, D), jnp.float32),
            pltpu.VMEM((CH, D), jnp.float32),
            pltpu.SemaphoreType.DMA,
            pltpu.SemaphoreType.DMA,
        ],
        compiler_params=pltpu.CompilerParams(needs_layout_passes=False),
    )
    def k(os_h, p0_h, p1_h, out_h,
          p0_v, p1_v, a0_v, b0_v, a1_v, b1_v, sem0, sem1):
        wid = lax.axis_index("s") * 2 + lax.axis_index("c")
        tokbase = wid * tpw
        pltpu.sync_copy(p0_h.at[pl.ds(tokbase, tpw)], p0_v)
        pltpu.sync_copy(p1_h.at[pl.ds(tokbase, tpw)], p1_v)
        abufs = (a0_v, a1_v)
        bbufs = (b0_v, b1_v)
        sems = (sem0, sem1)

        def fire(c, s):
            ca = pltpu.async_copy(os_h.at[p0_v.at[pl.ds(c * CH, CH)]],
                                  abufs[s], sems[s])
            cb = pltpu.async_copy(os_h.at[p1_v.at[pl.ds(c * CH, CH)]],
                                  bbufs[s], sems[s])
            return ca, cb

        cps = [None, None]
        cps[0] = fire(0, 0)
        for c in range(nch):
            s = c % 2
            if c + 1 < nch:
                cps[1 - s] = fire(c + 1, 1 - s)
            cps[s][0].wait()
            cps[s][1].wait()
            a_v, b_v = abufs[s], bbufs[s]

            def row(i, _):
                def col(j, _):
                    sl = pl.ds(j * 16, 16)
                    a_v[i, sl] = a_v[i, sl] + b_v[i, sl]
                    return 0
                lax.fori_loop(0, D // 16, col, 0)
                return 0
            lax.fori_loop(0, CH, row, 0)
            pltpu.sync_copy(a_v, out_h.at[pl.ds(tokbase + c * CH, CH)])

    return k(out_sorted, pos0, pos1)


def _final_body(x_ref, sh_ref, cb_ref, out_ref):
    out_ref[...] = x_ref[...] + sh_ref[...] + cb_ref[...]


def _final_add(x, shared, comb):
    blk = lambda: pl.BlockSpec((BTS, D), lambda i: (i, 0))
    return pl.pallas_call(
        _final_body,
        grid=(T // BTS,),
        in_specs=[blk(), blk(), blk()],
        out_specs=blk(),
        out_shape=jax.ShapeDtypeStruct((T, D), jnp.float32),
    )(x, shared, comb)


def kernel(x, Wg, bg, Wn, bn, Wr1, br1, Wr2, br2, Ws1, bs1, Ws2, bs2):
    eps = jax.random.normal(jax.random.key(42), (T, E), dtype=jnp.float32)
    i0, i1, r0, r1, g0, g1, cnt = _router(x, Wg, bg, Wn, bn, eps)
    prefix, block_expert = _dispatch_meta(cnt)
    shared = _shared_ffn(x, Ws1, bs1, Ws2, bs2)
    sorted_tok, sorted_gate, pos0, pos1 = _sc_scatter(
        i0, i1, r0, r1, g0, g1, prefix.reshape(NBLK * E))
    x_sorted = _sc_gather_x(x, sorted_tok)
    out_sorted = _grouped_ffn(block_expert, x_sorted, Wr1, br1, Wr2, br2,
                              sorted_gate.reshape(NB, 1, BT))
    comb = _sc_combine(out_sorted, pos0, pos1)
    return _final_add(x, shared, comb)
